# baseline (speedup 1.0000x reference)
import functools
import math

import jax
import jax.numpy as jnp
from jax import lax
from jax.experimental import pallas as pl
from jax.experimental.pallas import tpu as pltpu
from jax.experimental.pallas import tpu_sc as plsc

D_MODEL = 64
SCALE = math.sqrt(D_MODEL)
NC, NS, LANES = 2, 16, 16
NW = NC * NS

W = 128
SUB = 8
NBUF = 4


def _sc_embed(xT, lut_p):
    S, B0 = xT.shape
    n_blk = B0 // W
    bpw = n_blk // NW
    tpw = S * bpw
    mesh = plsc.VectorSubcoreMesh(core_axis_name="c", subcore_axis_name="s")

    @functools.partial(
        pl.kernel,
        out_type=jax.ShapeDtypeStruct((S, D_MODEL, B0), jnp.float32),
        mesh=mesh,
        compiler_params=pltpu.CompilerParams(needs_layout_passes=False),
        scratch_types=[
            pltpu.VMEM((S, bpw, W), jnp.int32),
            pltpu.VMEM((NBUF, W, W), jnp.float32),
            pltpu.VMEM((NBUF, D_MODEL // SUB, SUB, W), jnp.float32),
            [pltpu.SemaphoreType.DMA] * NBUF,
            [pltpu.SemaphoreType.DMA] * NBUF,
        ],
    )
    def k(x_hbm, lut_hbm, out_hbm, idx_v, rows_v, blk_v, gsems, wsems):
        wid = lax.axis_index("s") * NC + lax.axis_index("c")

        for ibl in range(bpw):
            pltpu.sync_copy(
                x_hbm.at[:, pl.ds((wid * bpw + ibl) * W, W)],
                idx_v.at[:, ibl],
            )

        iota = lax.iota(jnp.int32, LANES)
        d_hi = [2 * j + lax.shift_right_logical(iota, 3) for j in range(4)]
        d_lo = lax.bitwise_and(iota, SUB - 1)

        def task_sb(t):
            return t // bpw, t % bpw

        def gather_desc(t, b):
            s, ibl = task_sb(t)
            return pltpu.make_async_copy(
                lut_hbm.at[idx_v.at[s, ibl]], rows_v.at[b], gsems[b]
            )

        def write_descs(t, b):
            s, ibl = task_sb(t)
            col = (wid * bpw + ibl) * W
            return [
                pltpu.make_async_copy(
                    blk_v.at[b, dh],
                    out_hbm.at[s, pl.ds(dh * SUB, SUB), pl.ds(col, W)],
                    wsems[b],
                )
                for dh in range(D_MODEL // SUB)
            ]

        for b in range(2):
            gather_desc(b, b).start()

        @pl.loop(0, tpw // NBUF)
        def _(tt):
            t0 = tt * NBUF
            for b in range(NBUF):
                t = t0 + b
                pn = (b + 2) % NBUF

                @pl.when(t + 2 < tpw)
                def _():
                    @pl.when(t >= 2)
                    def _():
                        for d in write_descs(t - 2, pn):
                            d.wait()

                    gather_desc(t + 2, pn).start()

                gather_desc(t, b).wait()

                @plsc.parallel_loop(0, W, unroll=8)
                def _(i):
                    il = jnp.full((LANES,), i, jnp.int32)
                    for j in range(4):
                        v = rows_v[b, i, pl.ds(j * LANES, LANES)] * SCALE
                        plsc.store_scatter(
                            blk_v.at[b], [d_hi[j], d_lo, il], v
                        )

                for d in write_descs(t, b):
                    d.start()

        for d in write_descs(tpw - 2, (tpw - 2) % NBUF):
            d.wait()
        for d in write_descs(tpw - 1, (tpw - 1) % NBUF):
            d.wait()

    return k(xT, lut_p)


def _sc_format_table(lutT):
    D, V = lutT.shape
    n_full = V // W
    tail = V - n_full * W
    spw = -(-(-(-n_full // NW)) // NBUF) * NBUF
    mesh = plsc.VectorSubcoreMesh(core_axis_name="c", subcore_axis_name="s")

    @functools.partial(
        pl.kernel,
        out_type=jax.ShapeDtypeStruct((V, W), jnp.float32),
        mesh=mesh,
        compiler_params=pltpu.CompilerParams(needs_layout_passes=False),
        scratch_types=[
            pltpu.VMEM((NBUF, D_MODEL, W), jnp.float32),
            pltpu.VMEM((NBUF, W, W), jnp.float32),
            [pltpu.SemaphoreType.DMA] * NBUF,
            [pltpu.SemaphoreType.DMA] * NBUF,
        ],
    )
    def k(t_hbm, tail_hbm, out_hbm, sbuf, obuf, rsems, wsems):
        wid = lax.axis_index("s") * NC + lax.axis_index("c")

        iota = lax.iota(jnp.int32, LANES)
        vrow = [vj * LANES + iota for vj in range(W // LANES)]

        def slab_of(kk):
            return jnp.minimum(wid + kk * NW, n_full - 1)

        def read_desc(kk, b):
            return pltpu.make_async_copy(
                t_hbm.at[:, pl.ds(slab_of(kk) * W, W)], sbuf.at[b], rsems[b]
            )

        def write_desc(kk, b):
            return pltpu.make_async_copy(
                obuf.at[b],
                out_hbm.at[pl.ds(slab_of(kk) * W, W)],
                wsems[b],
            )

        def transpose(b, width):
            @plsc.parallel_loop(0, D_MODEL, unroll=8)
            def _(d):
                ds_ = jnp.full((LANES,), d, jnp.int32)
                for vj in range(width // LANES):
                    v = sbuf[b, d, pl.ds(vj * LANES, LANES)]
                    plsc.store_scatter(obuf.at[b], [vrow[vj], ds_], v)

        for b in range(2):
            read_desc(b, b).start()

        @pl.loop(0, spw // NBUF)
        def _(tt):
            k0 = tt * NBUF
            for b in range(NBUF):
                kk = k0 + b
                pn = (b + 2) % NBUF

                @pl.when(kk + 2 < spw)
                def _():
                    @pl.when(kk >= 2)
                    def _():
                        write_desc(kk - 2, pn).wait()

                    read_desc(kk + 2, pn).start()

                read_desc(kk, b).wait()
                transpose(b, W)
                write_desc(kk, b).start()

        write_desc(spw - 2, (spw - 2) % NBUF).wait()
        write_desc(spw - 1, (spw - 1) % NBUF).wait()

        if tail:

            @pl.when(wid == 0)
            def _():
                pltpu.sync_copy(tail_hbm, obuf.at[0, pl.ds(0, tail)])
                pltpu.sync_copy(
                    obuf.at[0, pl.ds(0, tail)],
                    out_hbm.at[pl.ds(n_full * W, tail)],
                )

    tail_rows = jax.lax.slice(jnp.transpose(lutT), (n_full * W, 0), (V, D))
    tail_rows = jnp.pad(tail_rows, ((0, 0), (0, W - D)))
    return k(lutT, tail_rows)


def kernel(x, lut):
    xT = jnp.transpose(x)
    lut_p = _sc_format_table(jnp.transpose(lut))
    outT = _sc_embed(xT, lut_p)
    return jnp.transpose(outT, (2, 0, 1))


# speedup vs baseline: 1.2780x; 1.2780x over previous
import functools
import math

import jax
import jax.numpy as jnp
from jax import lax
from jax.experimental import pallas as pl
from jax.experimental.pallas import tpu as pltpu
from jax.experimental.pallas import tpu_sc as plsc

D_MODEL = 64
SCALE = math.sqrt(D_MODEL)
NC, NS, LANES = 2, 16, 16
NW = NC * NS

W = 128
SUB = 8
NBUF = 4


def _sc_embed(xT, lut_p):
    S, B0 = xT.shape
    n_blk = B0 // W
    bpw = n_blk // NW
    tpw = S * bpw
    mesh = plsc.VectorSubcoreMesh(core_axis_name="c", subcore_axis_name="s")

    @functools.partial(
        pl.kernel,
        out_type=jax.ShapeDtypeStruct((S, D_MODEL, B0), jnp.float32),
        mesh=mesh,
        compiler_params=pltpu.CompilerParams(needs_layout_passes=False),
        scratch_types=[
            pltpu.VMEM((S, bpw, W), jnp.int32),
            pltpu.VMEM((NBUF, W, W), jnp.float32),
            pltpu.VMEM((NBUF, D_MODEL // SUB, SUB, W), jnp.float32),
            [pltpu.SemaphoreType.DMA] * NBUF,
            [pltpu.SemaphoreType.DMA] * NBUF,
        ],
    )
    def k(x_hbm, lut_hbm, out_hbm, idx_v, rows_v, blk_v, gsems, wsems):
        wid = lax.axis_index("s") * NC + lax.axis_index("c")

        for ibl in range(bpw):
            pltpu.sync_copy(
                x_hbm.at[:, pl.ds((wid * bpw + ibl) * W, W)],
                idx_v.at[:, ibl],
            )

        iota = lax.iota(jnp.int32, LANES)
        d_hi = [2 * j + lax.shift_right_logical(iota, 3) for j in range(4)]
        d_lo = lax.bitwise_and(iota, SUB - 1)

        def task_sb(t):
            return t // bpw, t % bpw

        def gather_desc(t, b):
            s, ibl = task_sb(t)
            return pltpu.make_async_copy(
                lut_hbm.at[idx_v.at[s, ibl]], rows_v.at[b], gsems[b]
            )

        def write_descs(t, b):
            s, ibl = task_sb(t)
            col = (wid * bpw + ibl) * W
            return [
                pltpu.make_async_copy(
                    blk_v.at[b, dh],
                    out_hbm.at[s, pl.ds(dh * SUB, SUB), pl.ds(col, W)],
                    wsems[b],
                )
                for dh in range(D_MODEL // SUB)
            ]

        for b in range(2):
            gather_desc(b, b).start()

        @pl.loop(0, tpw // NBUF)
        def _(tt):
            t0 = tt * NBUF
            for b in range(NBUF):
                t = t0 + b
                pn = (b + 2) % NBUF

                @pl.when(t + 2 < tpw)
                def _():
                    @pl.when(t >= 2)
                    def _():
                        for d in write_descs(t - 2, pn):
                            d.wait()

                    gather_desc(t + 2, pn).start()

                gather_desc(t, b).wait()

                @plsc.parallel_loop(0, W, unroll=8)
                def _(i):
                    il = jnp.full((LANES,), i, jnp.int32)
                    for j in range(4):
                        v = rows_v[b, i, pl.ds(j * LANES, LANES)] * SCALE
                        plsc.store_scatter(
                            blk_v.at[b], [d_hi[j], d_lo, il], v
                        )

                for d in write_descs(t, b):
                    d.start()

        for d in write_descs(tpw - 2, (tpw - 2) % NBUF):
            d.wait()
        for d in write_descs(tpw - 1, (tpw - 1) % NBUF):
            d.wait()

    return k(xT, lut_p)


def kernel(x, lut):
    xT = jnp.transpose(x)
    lut_p = jnp.pad(lut, ((0, 0), (0, W - D_MODEL)))
    outT = _sc_embed(xT, lut_p)
    return jnp.transpose(outT, (2, 0, 1))
